# baseline (speedup 1.0000x reference)
import jax
import jax.numpy as jnp
from jax import lax
from jax.experimental import pallas as pl
from jax.experimental.pallas import tpu as pltpu
from jax.experimental.pallas import tpu_sc as plsc

f32 = jnp.float32
i32 = jnp.int32

N = 10000
NP = 10240
D = 128
E = 320000
NC = 2
NS = 16
NW = NC * NS
EPW = E // NW
K = 40
NCH = EPW // K
SL = 5
RND = NCH // SL
KB = 80
SLB = 4
NCHB = EPW // KB
RNDB = 32
RPT = NP // NS
ZR = 40
DCH = 1024

_mesh = plsc.VectorSubcoreMesh(
    core_axis_name="c", subcore_axis_name="s", num_cores=NC, num_subcores=NS)


def _fill_zero_2d(ref, rows):
    def body(r, c):
        for cc in range(D // 16):
            ref[r, pl.ds(cc * 16, 16)] = jnp.zeros((16,), f32)
        return c
    lax.fori_loop(0, rows, body, 0)


def _fill_zero_1d(ref, n):
    def body(i, c):
        ref[pl.ds(i * 16, 16)] = jnp.zeros((16,), f32)
        return c
    lax.fori_loop(0, n // 16, body, 0)


def _fwd_body(h_hbm, ei_hbm, outp, degd_o, degs_o,
              acc_sh, dd_sh, ds_sh, sidx, didx, rows, ones_v,
              zbuf, zdeg, isem, gsem, ssem, zsem):
    cid = lax.axis_index("c")
    sid = lax.axis_index("s")
    wid = sid * NC + cid
    base = wid * EPW

    for i in range(SL):
        off = base + i * K
        pltpu.async_copy(ei_hbm.at[pl.ds(off, K)], sidx[i], isem[i])
        pltpu.async_copy(ei_hbm.at[pl.ds(E + off, K)], didx[i], isem[i])

    _fill_zero_2d(zbuf, ZR)
    _fill_zero_1d(zdeg, DCH)
    for i in range(48 // 16):
        ones_v[pl.ds(i * 16, 16)] = jnp.ones((16,), f32)
    ones_s = ones_v.at[pl.ds(0, K)]

    for b in range(RPT // ZR):
        pltpu.async_copy(zbuf, acc_sh.at[pl.ds(sid * RPT + b * ZR, ZR)], zsem)

    @pl.when(sid < 10)
    def _():
        pltpu.async_copy(zdeg, dd_sh.at[pl.ds(sid * DCH, DCH)], zsem)
        pltpu.async_copy(zdeg, ds_sh.at[pl.ds(sid * DCH, DCH)], zsem)

    def wait_idx(i):
        pltpu.make_async_copy(ei_hbm.at[pl.ds(0, K)], sidx[i], isem[i]).wait()

    def wait_rows(i, sem):
        pltpu.make_async_copy(h_hbm.at[pl.ds(0, K)], rows[i], sem[i]).wait()

    for i in range(SL):
        wait_idx(i)
        wait_idx(i)
        pltpu.async_copy(h_hbm.at[sidx[i]], rows[i], gsem[i])

    for b in range(RPT // ZR):
        pltpu.make_async_copy(h_hbm.at[pl.ds(0, ZR)], zbuf, zsem).wait()

    @pl.when(sid < 10)
    def _():
        pltpu.make_async_copy(degd_o.at[pl.ds(0, DCH)], zdeg, zsem).wait()
        pltpu.make_async_copy(degd_o.at[pl.ds(0, DCH)], zdeg, zsem).wait()

    plsc.subcore_barrier()

    def wait_deg(i):
        pltpu.make_async_copy(degd_o.at[pl.ds(0, K)], ones_s, ssem[i]).wait()

    def round_(j, c):
        for i in range(SL):
            wait_rows(i, gsem)
            pltpu.async_copy(ones_s, dd_sh.at[didx[i]], ssem[i], add=True)
            pltpu.async_copy(ones_s, ds_sh.at[sidx[i]], ssem[i], add=True)
            pltpu.async_copy(rows[i], acc_sh.at[didx[i]], ssem[i], add=True)

        @pl.when(j < RND - 1)
        def _():
            for i in range(SL):
                wait_deg(i)
                wait_deg(i)
                wait_rows(i, ssem)
                off = base + ((j + 1) * SL + i) * K
                pltpu.async_copy(ei_hbm.at[pl.ds(off, K)], sidx[i], isem[i])
                pltpu.async_copy(ei_hbm.at[pl.ds(E + off, K)], didx[i],
                                 isem[i])
            for i in range(SL):
                wait_idx(i)
                wait_idx(i)
                pltpu.async_copy(h_hbm.at[sidx[i]], rows[i], gsem[i])
        return c

    lax.fori_loop(0, RND, round_, 0)
    for i in range(SL):
        wait_deg(i)
        wait_deg(i)
        wait_rows(i, ssem)

    plsc.subcore_barrier()

    pltpu.sync_copy(acc_sh.at[pl.ds(sid * RPT, RPT)],
                    outp.at[cid, pl.ds(sid * RPT, RPT)])

    @pl.when(sid < 10)
    def _():
        pltpu.sync_copy(dd_sh.at[pl.ds(sid * DCH, DCH)],
                        degd_o.at[pl.ds(cid * NP + sid * DCH, DCH)])
        pltpu.sync_copy(ds_sh.at[pl.ds(sid * DCH, DCH)],
                        degs_o.at[pl.ds(cid * NP + sid * DCH, DCH)])


_fwd = pl.kernel(
    _fwd_body,
    out_type=(
        jax.ShapeDtypeStruct((NC, NP, D), f32),
        jax.ShapeDtypeStruct((NC * NP,), f32),
        jax.ShapeDtypeStruct((NC * NP,), f32),
    ),
    mesh=_mesh,
    scratch_types=[
        pltpu.VMEM_SHARED((NP, D), f32),
        pltpu.VMEM_SHARED((NP,), f32),
        pltpu.VMEM_SHARED((NP,), f32),
        [pltpu.VMEM((K,), i32)] * SL,
        [pltpu.VMEM((K,), i32)] * SL,
        [pltpu.VMEM((K, D), f32)] * SL,
        pltpu.VMEM((48,), f32),
        pltpu.VMEM((ZR, D), f32),
        pltpu.VMEM((DCH,), f32),
        [pltpu.SemaphoreType.DMA] * SL,
        [pltpu.SemaphoreType.DMA] * SL,
        [pltpu.SemaphoreType.DMA] * SL,
        pltpu.SemaphoreType.DMA,
    ],
)


def _bwd_body(r_hbm, ei_hbm, outq,
              acc_sh, sidx, didx, rows, zbuf,
              isem, gsem, ssem, zsem):
    cid = lax.axis_index("c")
    sid = lax.axis_index("s")
    wid = sid * NC + cid
    base = wid * EPW

    for i in range(SLB):
        off = base + i * KB
        pltpu.async_copy(ei_hbm.at[pl.ds(off, KB)], sidx[i], isem[i])
        pltpu.async_copy(ei_hbm.at[pl.ds(E + off, KB)], didx[i], isem[i])

    _fill_zero_2d(zbuf, ZR)
    for b in range(RPT // ZR):
        pltpu.async_copy(zbuf, acc_sh.at[pl.ds(sid * RPT + b * ZR, ZR)], zsem)

    def wait_idx(i):
        pltpu.make_async_copy(ei_hbm.at[pl.ds(0, KB)], sidx[i], isem[i]).wait()

    def wait_rows(i, sem):
        pltpu.make_async_copy(r_hbm.at[pl.ds(0, KB)], rows[i], sem[i]).wait()

    for i in range(SLB):
        wait_idx(i)
        wait_idx(i)
        pltpu.async_copy(r_hbm.at[didx[i]], rows[i], gsem[i])

    for b in range(RPT // ZR):
        pltpu.make_async_copy(r_hbm.at[pl.ds(0, ZR)], zbuf, zsem).wait()
    plsc.subcore_barrier()

    def round_(j, c):
        for i in range(SLB):
            ch = j * SLB + i

            @pl.when(ch < NCHB)
            def _():
                wait_rows(i, gsem)
                pltpu.async_copy(rows[i], acc_sh.at[sidx[i]], ssem[i],
                                 add=True)

        @pl.when(j < RNDB - 1)
        def _():
            for i in range(SLB):
                ch2 = (j + 1) * SLB + i

                @pl.when(ch2 < NCHB)
                def _():
                    wait_rows(i, ssem)
                    off = base + ch2 * KB
                    pltpu.async_copy(ei_hbm.at[pl.ds(off, KB)], sidx[i],
                                     isem[i])
                    pltpu.async_copy(ei_hbm.at[pl.ds(E + off, KB)], didx[i],
                                     isem[i])
            for i in range(SLB):
                ch2 = (j + 1) * SLB + i

                @pl.when(ch2 < NCHB)
                def _():
                    wait_idx(i)
                    wait_idx(i)
                    pltpu.async_copy(r_hbm.at[didx[i]], rows[i], gsem[i])
        return c

    lax.fori_loop(0, RNDB, round_, 0)
    for i in range(SLB):
        wait_rows(i, ssem)
    plsc.subcore_barrier()

    pltpu.sync_copy(acc_sh.at[pl.ds(sid * RPT, RPT)],
                    outq.at[cid, pl.ds(sid * RPT, RPT)])


_bwd = pl.kernel(
    _bwd_body,
    out_type=jax.ShapeDtypeStruct((NC, NP, D), f32),
    mesh=_mesh,
    scratch_types=[
        pltpu.VMEM_SHARED((NP, D), f32),
        [pltpu.VMEM((KB,), i32)] * SLB,
        [pltpu.VMEM((KB,), i32)] * SLB,
        [pltpu.VMEM((KB, D), f32)] * SLB,
        pltpu.VMEM((ZR, D), f32),
        [pltpu.SemaphoreType.DMA] * SLB,
        [pltpu.SemaphoreType.DMA] * SLB,
        [pltpu.SemaphoreType.DMA] * SLB,
        pltpu.SemaphoreType.DMA,
    ],
)

RB = 1000


def _comb_body(n2_ref, pp_ref, dd_ref, o_ref):
    s = pp_ref[0] + pp_ref[1]
    deg = jnp.clip(dd_ref[0] + dd_ref[1], 1.0, None)
    o_ref[...] = s * jnp.power(deg, n2_ref[0])


def _combine(pp, dd, n2):
    dd3 = dd.reshape(NC, NP, 1)
    return pl.pallas_call(
        _comb_body,
        grid=(N // RB,),
        in_specs=[
            pl.BlockSpec(memory_space=pltpu.SMEM),
            pl.BlockSpec((NC, RB, D), lambda i: (0, i, 0)),
            pl.BlockSpec((NC, RB, 1), lambda i: (0, i, 0)),
        ],
        out_specs=pl.BlockSpec((RB, D), lambda i: (i, 0)),
        out_shape=jax.ShapeDtypeStruct((N, D), f32),
    )(n2, pp, dd3)


def kernel(h_src, h_dst, edge_index, norm_2):
    del h_dst
    ei = edge_index.astype(i32).reshape(2 * E)
    h = h_src.astype(f32)
    pp, ddp, dsp = _fwd(h, ei)
    rst = _combine(pp, ddp, jnp.full((1,), -1.0, f32))
    qq = _bwd(rst, ei)
    bsrc = _combine(qq, dsp, jnp.asarray(norm_2, f32).reshape(1))
    return (bsrc, rst)


# speedup vs baseline: 1.0116x; 1.0116x over previous
import jax
import jax.numpy as jnp
from jax import lax
from jax.experimental import pallas as pl
from jax.experimental.pallas import tpu as pltpu
from jax.experimental.pallas import tpu_sc as plsc

f32 = jnp.float32
i32 = jnp.int32

N = 10000
NP = 10240
D = 128
E = 320000
NC = 2
NS = 16
NW = NC * NS
EPW = E // NW
K = 40
NCH = EPW // K
SL = 5
RND = NCH // SL
KB = 80
SLB = 4
NCHB = EPW // KB
RNDB = 32
RPT = NP // NS
ZR = 40
DCH = 1024

_mesh = plsc.VectorSubcoreMesh(
    core_axis_name="c", subcore_axis_name="s", num_cores=NC, num_subcores=NS)


def _fill_zero_2d(ref, rows):
    def body(r, c):
        for cc in range(D // 16):
            ref[r, pl.ds(cc * 16, 16)] = jnp.zeros((16,), f32)
        return c
    lax.fori_loop(0, rows, body, 0)


def _fill_zero_1d(ref, n):
    def body(i, c):
        ref[pl.ds(i * 16, 16)] = jnp.zeros((16,), f32)
        return c
    lax.fori_loop(0, n // 16, body, 0)


def _fwd_body(h_hbm, ei_hbm, outp, degd_o, degs_o,
              acc_sh, dd_sh, ds_sh, sidx, didx, rows, ones_v,
              zbuf, zdeg, isem, gsem, ssem, zsem):
    cid = lax.axis_index("c")
    sid = lax.axis_index("s")
    wid = sid * NC + cid
    base = wid * EPW

    for i in range(SL):
        off = base + i * K
        pltpu.async_copy(ei_hbm.at[pl.ds(off, K)], sidx[i], isem[i])
        pltpu.async_copy(ei_hbm.at[pl.ds(E + off, K)], didx[i], isem[i])

    _fill_zero_2d(zbuf, ZR)
    _fill_zero_1d(zdeg, DCH)
    for i in range(48 // 16):
        ones_v[pl.ds(i * 16, 16)] = jnp.ones((16,), f32)
    ones_s = ones_v.at[pl.ds(0, K)]

    for b in range(RPT // ZR):
        pltpu.async_copy(zbuf, acc_sh.at[pl.ds(sid * RPT + b * ZR, ZR)], zsem)

    @pl.when(sid < 10)
    def _():
        pltpu.async_copy(zdeg, dd_sh.at[pl.ds(sid * DCH, DCH)], zsem)
        pltpu.async_copy(zdeg, ds_sh.at[pl.ds(sid * DCH, DCH)], zsem)

    def wait_idx(i):
        pltpu.make_async_copy(ei_hbm.at[pl.ds(0, K)], sidx[i], isem[i]).wait()

    def wait_rows(i, sem):
        pltpu.make_async_copy(h_hbm.at[pl.ds(0, K)], rows[i], sem[i]).wait()

    for i in range(SL):
        wait_idx(i)
        wait_idx(i)
        pltpu.async_copy(h_hbm.at[sidx[i]], rows[i], gsem[i])

    for b in range(RPT // ZR):
        pltpu.make_async_copy(h_hbm.at[pl.ds(0, ZR)], zbuf, zsem).wait()

    @pl.when(sid < 10)
    def _():
        pltpu.make_async_copy(degd_o.at[pl.ds(0, DCH)], zdeg, zsem).wait()
        pltpu.make_async_copy(degd_o.at[pl.ds(0, DCH)], zdeg, zsem).wait()

    plsc.subcore_barrier()

    def wait_deg(i):
        pltpu.make_async_copy(degd_o.at[pl.ds(0, K)], ones_s, ssem[i]).wait()

    def round_(j, c):
        for i in range(SL):
            wait_rows(i, gsem)
            pltpu.async_copy(ones_s, dd_sh.at[didx[i]], ssem[i], add=True)
            pltpu.async_copy(ones_s, ds_sh.at[sidx[i]], ssem[i], add=True)
            pltpu.async_copy(rows[i], acc_sh.at[didx[i]], ssem[i], add=True)

        @pl.when(j < RND - 1)
        def _():
            for i in range(SL):
                wait_deg(i)
                wait_deg(i)
                wait_rows(i, ssem)
                off = base + ((j + 1) * SL + i) * K
                pltpu.async_copy(ei_hbm.at[pl.ds(off, K)], sidx[i], isem[i])
                pltpu.async_copy(ei_hbm.at[pl.ds(E + off, K)], didx[i],
                                 isem[i])
            for i in range(SL):
                wait_idx(i)
                wait_idx(i)
                pltpu.async_copy(h_hbm.at[sidx[i]], rows[i], gsem[i])
        return c

    lax.fori_loop(0, RND, round_, 0)
    for i in range(SL):
        wait_deg(i)
        wait_deg(i)
        wait_rows(i, ssem)

    plsc.subcore_barrier()

    pltpu.sync_copy(acc_sh.at[pl.ds(sid * RPT, RPT)],
                    outp.at[cid, pl.ds(sid * RPT, RPT)])

    @pl.when(sid < 10)
    def _():
        pltpu.sync_copy(dd_sh.at[pl.ds(sid * DCH, DCH)],
                        degd_o.at[pl.ds(cid * NP + sid * DCH, DCH)])
        pltpu.sync_copy(ds_sh.at[pl.ds(sid * DCH, DCH)],
                        degs_o.at[pl.ds(cid * NP + sid * DCH, DCH)])


_fwd = pl.kernel(
    _fwd_body,
    out_type=(
        jax.ShapeDtypeStruct((NC, NP, D), f32),
        jax.ShapeDtypeStruct((NC * NP,), f32),
        jax.ShapeDtypeStruct((NC * NP,), f32),
    ),
    mesh=_mesh,
    scratch_types=[
        pltpu.VMEM_SHARED((NP, D), f32),
        pltpu.VMEM_SHARED((NP,), f32),
        pltpu.VMEM_SHARED((NP,), f32),
        [pltpu.VMEM((K,), i32)] * SL,
        [pltpu.VMEM((K,), i32)] * SL,
        [pltpu.VMEM((K, D), f32)] * SL,
        pltpu.VMEM((48,), f32),
        pltpu.VMEM((ZR, D), f32),
        pltpu.VMEM((DCH,), f32),
        [pltpu.SemaphoreType.DMA] * SL,
        [pltpu.SemaphoreType.DMA] * SL,
        [pltpu.SemaphoreType.DMA] * SL,
        pltpu.SemaphoreType.DMA,
    ],
)


def _bwd_body(r_hbm, ei_hbm, outq,
              acc_sh, sidx, didx, rows, zbuf,
              isem, gsem, ssem, zsem):
    cid = lax.axis_index("c")
    sid = lax.axis_index("s")
    wid = sid * NC + cid
    base = wid * EPW

    for i in range(SLB):
        off = base + i * KB
        pltpu.async_copy(ei_hbm.at[pl.ds(off, KB)], sidx[i], isem[i])
        pltpu.async_copy(ei_hbm.at[pl.ds(E + off, KB)], didx[i], isem[i])

    _fill_zero_2d(zbuf, ZR)
    for b in range(RPT // ZR):
        pltpu.async_copy(zbuf, acc_sh.at[pl.ds(sid * RPT + b * ZR, ZR)], zsem)

    def wait_idx(i):
        pltpu.make_async_copy(ei_hbm.at[pl.ds(0, KB)], sidx[i], isem[i]).wait()

    def wait_rows(i, sem):
        pltpu.make_async_copy(r_hbm.at[pl.ds(0, KB)], rows[i], sem[i]).wait()

    for i in range(SLB):
        wait_idx(i)
        wait_idx(i)
        pltpu.async_copy(r_hbm.at[didx[i]], rows[i], gsem[i])

    for b in range(RPT // ZR):
        pltpu.make_async_copy(r_hbm.at[pl.ds(0, ZR)], zbuf, zsem).wait()
    plsc.subcore_barrier()

    def round_(j, c):
        for i in range(SLB):
            ch = j * SLB + i

            @pl.when(ch < NCHB)
            def _():
                wait_rows(i, gsem)
                pltpu.async_copy(rows[i], acc_sh.at[sidx[i]], ssem[i],
                                 add=True)

        @pl.when(j < RNDB - 1)
        def _():
            for i in range(SLB):
                ch2 = (j + 1) * SLB + i

                @pl.when(ch2 < NCHB)
                def _():
                    wait_rows(i, ssem)
                    off = base + ch2 * KB
                    pltpu.async_copy(ei_hbm.at[pl.ds(off, KB)], sidx[i],
                                     isem[i])
                    pltpu.async_copy(ei_hbm.at[pl.ds(E + off, KB)], didx[i],
                                     isem[i])
            for i in range(SLB):
                ch2 = (j + 1) * SLB + i

                @pl.when(ch2 < NCHB)
                def _():
                    wait_idx(i)
                    wait_idx(i)
                    pltpu.async_copy(r_hbm.at[didx[i]], rows[i], gsem[i])
        return c

    lax.fori_loop(0, RNDB, round_, 0)
    for i in range(SLB):
        wait_rows(i, ssem)
    plsc.subcore_barrier()

    pltpu.sync_copy(acc_sh.at[pl.ds(sid * RPT, RPT)],
                    outq.at[cid, pl.ds(sid * RPT, RPT)])


_bwd = pl.kernel(
    _bwd_body,
    out_type=jax.ShapeDtypeStruct((NC, NP, D), f32),
    mesh=_mesh,
    scratch_types=[
        pltpu.VMEM_SHARED((NP, D), f32),
        [pltpu.VMEM((KB,), i32)] * SLB,
        [pltpu.VMEM((KB,), i32)] * SLB,
        [pltpu.VMEM((KB, D), f32)] * SLB,
        pltpu.VMEM((ZR, D), f32),
        [pltpu.SemaphoreType.DMA] * SLB,
        [pltpu.SemaphoreType.DMA] * SLB,
        [pltpu.SemaphoreType.DMA] * SLB,
        pltpu.SemaphoreType.DMA,
    ],
)

RB = 2000


def _comb_body(n2_ref, pp_ref, dd_ref, o_ref):
    s = pp_ref[0] + pp_ref[1]
    deg = jnp.clip(dd_ref[0] + dd_ref[1], 1.0, None)
    o_ref[...] = s * jnp.power(deg, n2_ref[0])


def _combine(pp, dd, n2):
    dd3 = dd.reshape(NC, NP, 1)
    return pl.pallas_call(
        _comb_body,
        grid=(N // RB,),
        in_specs=[
            pl.BlockSpec(memory_space=pltpu.SMEM),
            pl.BlockSpec((NC, RB, D), lambda i: (0, i, 0)),
            pl.BlockSpec((NC, RB, 1), lambda i: (0, i, 0)),
        ],
        out_specs=pl.BlockSpec((RB, D), lambda i: (i, 0)),
        out_shape=jax.ShapeDtypeStruct((N, D), f32),
    )(n2, pp, dd3)


def kernel(h_src, h_dst, edge_index, norm_2):
    del h_dst
    ei = edge_index.astype(i32).reshape(2 * E)
    h = h_src.astype(f32)
    pp, ddp, dsp = _fwd(h, ei)
    rst = _combine(pp, ddp, jnp.full((1,), -1.0, f32))
    qq = _bwd(rst, ei)
    bsrc = _combine(qq, dsp, jnp.asarray(norm_2, f32).reshape(1))
    return (bsrc, rst)


# speedup vs baseline: 1.0263x; 1.0145x over previous
import jax
import jax.numpy as jnp
from jax import lax
from jax.experimental import pallas as pl
from jax.experimental.pallas import tpu as pltpu
from jax.experimental.pallas import tpu_sc as plsc

f32 = jnp.float32
i32 = jnp.int32

N = 10000
NP = 10240
D = 128
E = 320000
NC = 2
NS = 16
NW = NC * NS
EPW = E // NW
K = 40
NCH = EPW // K
SL = 5
RND = NCH // SL
KB = 80
SLB = 4
NCHB = EPW // KB
RNDB = 32
RPT = NP // NS
ZR = 40
DCH = 1024

_mesh = plsc.VectorSubcoreMesh(
    core_axis_name="c", subcore_axis_name="s", num_cores=NC, num_subcores=NS)


def _fill_zero_2d(ref, rows):
    def body(r, c):
        for cc in range(D // 16):
            ref[r, pl.ds(cc * 16, 16)] = jnp.zeros((16,), f32)
        return c
    lax.fori_loop(0, rows, body, 0)


def _fill_zero_1d(ref, n):
    def body(i, c):
        ref[pl.ds(i * 16, 16)] = jnp.zeros((16,), f32)
        return c
    lax.fori_loop(0, n // 16, body, 0)


def _fwd_body(h_hbm, ei_hbm, outp, degd_o, degs_o,
              acc_sh, dd_sh, ds_sh, sidx, didx, rows, ones_v,
              zbuf, zdeg, isem, gsem, ssem, zsem):
    cid = lax.axis_index("c")
    sid = lax.axis_index("s")
    wid = sid * NC + cid
    base = wid * EPW

    for i in range(SL):
        off = base + i * K
        pltpu.async_copy(ei_hbm.at[pl.ds(off, K)], sidx[i], isem[i])
        pltpu.async_copy(ei_hbm.at[pl.ds(E + off, K)], didx[i], isem[i])

    _fill_zero_2d(zbuf, ZR)
    _fill_zero_1d(zdeg, DCH)
    for i in range(48 // 16):
        ones_v[pl.ds(i * 16, 16)] = jnp.ones((16,), f32)
    ones_s = ones_v.at[pl.ds(0, K)]

    for b in range(RPT // ZR):
        pltpu.async_copy(zbuf, acc_sh.at[pl.ds(sid * RPT + b * ZR, ZR)], zsem)

    @pl.when(sid < 10)
    def _():
        pltpu.async_copy(zdeg, dd_sh.at[pl.ds(sid * DCH, DCH)], zsem)
        pltpu.async_copy(zdeg, ds_sh.at[pl.ds(sid * DCH, DCH)], zsem)

    def wait_idx(i):
        pltpu.make_async_copy(ei_hbm.at[pl.ds(0, K)], sidx[i], isem[i]).wait()

    def wait_rows(i, sem):
        pltpu.make_async_copy(h_hbm.at[pl.ds(0, K)], rows[i], sem[i]).wait()

    for i in range(SL):
        wait_idx(i)
        wait_idx(i)
        pltpu.async_copy(h_hbm.at[sidx[i]], rows[i], gsem[i])

    for b in range(RPT // ZR):
        pltpu.make_async_copy(h_hbm.at[pl.ds(0, ZR)], zbuf, zsem).wait()

    @pl.when(sid < 10)
    def _():
        pltpu.make_async_copy(degd_o.at[pl.ds(0, DCH)], zdeg, zsem).wait()
        pltpu.make_async_copy(degd_o.at[pl.ds(0, DCH)], zdeg, zsem).wait()

    plsc.subcore_barrier()

    def wait_deg(i):
        pltpu.make_async_copy(degd_o.at[pl.ds(0, K)], ones_s, ssem[i]).wait()

    def round_(j, c):
        for i in range(SL):
            wait_rows(i, gsem)
            pltpu.async_copy(ones_s, dd_sh.at[didx[i]], ssem[i], add=True)
            pltpu.async_copy(ones_s, ds_sh.at[sidx[i]], ssem[i], add=True)
            pltpu.async_copy(rows[i], acc_sh.at[didx[i]], ssem[i], add=True)

        @pl.when(j < RND - 1)
        def _():
            for i in range(SL):
                wait_deg(i)
                wait_deg(i)
                wait_rows(i, ssem)
                off = base + ((j + 1) * SL + i) * K
                pltpu.async_copy(ei_hbm.at[pl.ds(off, K)], sidx[i], isem[i])
                pltpu.async_copy(ei_hbm.at[pl.ds(E + off, K)], didx[i],
                                 isem[i])
            for i in range(SL):
                wait_idx(i)
                wait_idx(i)
                pltpu.async_copy(h_hbm.at[sidx[i]], rows[i], gsem[i])
        return c

    lax.fori_loop(0, RND, round_, 0)
    for i in range(SL):
        wait_deg(i)
        wait_deg(i)
        wait_rows(i, ssem)

    plsc.subcore_barrier()

    pltpu.sync_copy(acc_sh.at[pl.ds(sid * RPT, RPT)],
                    outp.at[cid, pl.ds(sid * RPT, RPT)])

    @pl.when(sid < 10)
    def _():
        pltpu.sync_copy(dd_sh.at[pl.ds(sid * DCH, DCH)],
                        degd_o.at[pl.ds(cid * NP + sid * DCH, DCH)])
        pltpu.sync_copy(ds_sh.at[pl.ds(sid * DCH, DCH)],
                        degs_o.at[pl.ds(cid * NP + sid * DCH, DCH)])


_fwd = pl.kernel(
    _fwd_body,
    out_type=(
        jax.ShapeDtypeStruct((NC, NP, D), f32),
        jax.ShapeDtypeStruct((NC * NP,), f32),
        jax.ShapeDtypeStruct((NC * NP,), f32),
    ),
    mesh=_mesh,
    scratch_types=[
        pltpu.VMEM_SHARED((NP, D), f32),
        pltpu.VMEM_SHARED((NP,), f32),
        pltpu.VMEM_SHARED((NP,), f32),
        [pltpu.VMEM((K,), i32)] * SL,
        [pltpu.VMEM((K,), i32)] * SL,
        [pltpu.VMEM((K, D), f32)] * SL,
        pltpu.VMEM((48,), f32),
        pltpu.VMEM((ZR, D), f32),
        pltpu.VMEM((DCH,), f32),
        [pltpu.SemaphoreType.DMA] * SL,
        [pltpu.SemaphoreType.DMA] * SL,
        [pltpu.SemaphoreType.DMA] * SL,
        pltpu.SemaphoreType.DMA,
    ],
)


def _bwd_body(r_hbm, ei_hbm, outq,
              acc_sh, sidx, didx, rows, zbuf,
              isem, gsem, ssem, zsem):
    cid = lax.axis_index("c")
    sid = lax.axis_index("s")
    wid = sid * NC + cid
    base = wid * EPW

    for i in range(SLB):
        off = base + i * KB
        pltpu.async_copy(ei_hbm.at[pl.ds(off, KB)], sidx[i], isem[i])
        pltpu.async_copy(ei_hbm.at[pl.ds(E + off, KB)], didx[i], isem[i])

    _fill_zero_2d(zbuf, ZR)
    for b in range(RPT // ZR):
        pltpu.async_copy(zbuf, acc_sh.at[pl.ds(sid * RPT + b * ZR, ZR)], zsem)

    def wait_idx(i):
        pltpu.make_async_copy(ei_hbm.at[pl.ds(0, KB)], sidx[i], isem[i]).wait()

    def wait_rows(i, sem):
        pltpu.make_async_copy(r_hbm.at[pl.ds(0, KB)], rows[i], sem[i]).wait()

    for i in range(SLB):
        wait_idx(i)
        wait_idx(i)
        pltpu.async_copy(r_hbm.at[didx[i]], rows[i], gsem[i])

    for b in range(RPT // ZR):
        pltpu.make_async_copy(r_hbm.at[pl.ds(0, ZR)], zbuf, zsem).wait()
    plsc.subcore_barrier()

    def round_(j, c):
        for i in range(SLB):
            ch = j * SLB + i

            @pl.when(ch < NCHB)
            def _():
                wait_rows(i, gsem)
                pltpu.async_copy(rows[i], acc_sh.at[sidx[i]], ssem[i],
                                 add=True)

        @pl.when(j < RNDB - 1)
        def _():
            for i in range(SLB):
                ch2 = (j + 1) * SLB + i

                @pl.when(ch2 < NCHB)
                def _():
                    wait_rows(i, ssem)
                    off = base + ch2 * KB
                    pltpu.async_copy(ei_hbm.at[pl.ds(off, KB)], sidx[i],
                                     isem[i])
                    pltpu.async_copy(ei_hbm.at[pl.ds(E + off, KB)], didx[i],
                                     isem[i])
            for i in range(SLB):
                ch2 = (j + 1) * SLB + i

                @pl.when(ch2 < NCHB)
                def _():
                    wait_idx(i)
                    wait_idx(i)
                    pltpu.async_copy(r_hbm.at[didx[i]], rows[i], gsem[i])
        return c

    lax.fori_loop(0, RNDB, round_, 0)
    for i in range(SLB):
        wait_rows(i, ssem)
    plsc.subcore_barrier()

    pltpu.sync_copy(acc_sh.at[pl.ds(sid * RPT, RPT)],
                    outq.at[cid, pl.ds(sid * RPT, RPT)])


_bwd = pl.kernel(
    _bwd_body,
    out_type=jax.ShapeDtypeStruct((NC, NP, D), f32),
    mesh=_mesh,
    scratch_types=[
        pltpu.VMEM_SHARED((NP, D), f32),
        [pltpu.VMEM((KB,), i32)] * SLB,
        [pltpu.VMEM((KB,), i32)] * SLB,
        [pltpu.VMEM((KB, D), f32)] * SLB,
        pltpu.VMEM((ZR, D), f32),
        [pltpu.SemaphoreType.DMA] * SLB,
        [pltpu.SemaphoreType.DMA] * SLB,
        [pltpu.SemaphoreType.DMA] * SLB,
        pltpu.SemaphoreType.DMA,
    ],
)

RB = 2000


def _comb_body(n2_ref, pp_ref, dd_ref, o_ref):
    s = pp_ref[0] + pp_ref[1]
    deg = jnp.clip(dd_ref[0] + dd_ref[1], 1.0, None)
    o_ref[...] = s * jnp.power(deg, n2_ref[0])


def _combine(pp, dd, n2):
    dd3 = dd.reshape(NC, NP)[:, :N].reshape(NC, N, 1)
    return pl.pallas_call(
        _comb_body,
        grid=(N // RB,),
        in_specs=[
            pl.BlockSpec(memory_space=pltpu.SMEM),
            pl.BlockSpec((NC, RB, D), lambda i: (0, i, 0)),
            pl.BlockSpec((NC, RB, 1), lambda i: (0, i, 0)),
        ],
        out_specs=pl.BlockSpec((RB, D), lambda i: (i, 0)),
        out_shape=jax.ShapeDtypeStruct((N, D), f32),
    )(n2, pp, dd3)


def kernel(h_src, h_dst, edge_index, norm_2):
    del h_dst
    ei = edge_index.astype(i32).reshape(2 * E)
    h = h_src.astype(f32)
    pp, ddp, dsp = _fwd(h, ei)
    rst = _combine(pp, ddp, jnp.full((1,), -1.0, f32))
    qq = _bwd(rst, ei)
    bsrc = _combine(qq, dsp, jnp.asarray(norm_2, f32).reshape(1))
    return (bsrc, rst)


# speedup vs baseline: 1.0312x; 1.0047x over previous
import jax
import jax.numpy as jnp
from jax import lax
from jax.experimental import pallas as pl
from jax.experimental.pallas import tpu as pltpu
from jax.experimental.pallas import tpu_sc as plsc

f32 = jnp.float32
i32 = jnp.int32

N = 10000
NP = 10240
D = 128
E = 320000
NC = 2
NS = 16
NW = NC * NS
EPW = E // NW
K = 40
NCH = EPW // K
SL = 5
RND = NCH // SL
KB = 80
SLB = 4
NCHB = EPW // KB
RNDB = 32
RPT = NP // NS
ZR = 40
DCH = 1024

_mesh = plsc.VectorSubcoreMesh(
    core_axis_name="c", subcore_axis_name="s", num_cores=NC, num_subcores=NS)


def _fill_zero_2d(ref, rows):
    def body(r, c):
        for cc in range(D // 16):
            ref[r, pl.ds(cc * 16, 16)] = jnp.zeros((16,), f32)
        return c
    lax.fori_loop(0, rows, body, 0)


def _fill_zero_1d(ref, n):
    def body(i, c):
        ref[pl.ds(i * 16, 16)] = jnp.zeros((16,), f32)
        return c
    lax.fori_loop(0, n // 16, body, 0)


def _fwd_body(h_hbm, ei_hbm, outp, degd_o, degs_o,
              acc_sh, dd_sh, ds_sh, sidx, didx, rows, ones_v,
              zbuf, zdeg, isem, gsem, ssem, zsem):
    cid = lax.axis_index("c")
    sid = lax.axis_index("s")
    wid = sid * NC + cid
    base = wid * EPW

    for i in range(SL):
        off = base + i * K
        pltpu.async_copy(ei_hbm.at[pl.ds(off, K)], sidx[i], isem[i])
        pltpu.async_copy(ei_hbm.at[pl.ds(E + off, K)], didx[i], isem[i])

    _fill_zero_2d(zbuf, ZR)
    _fill_zero_1d(zdeg, DCH)
    for i in range(48 // 16):
        ones_v[pl.ds(i * 16, 16)] = jnp.ones((16,), f32)
    ones_s = ones_v.at[pl.ds(0, K)]

    for b in range(RPT // ZR):
        pltpu.async_copy(zbuf, acc_sh.at[pl.ds(sid * RPT + b * ZR, ZR)], zsem)

    @pl.when(sid < 10)
    def _():
        pltpu.async_copy(zdeg, dd_sh.at[pl.ds(sid * DCH, DCH)], zsem)
        pltpu.async_copy(zdeg, ds_sh.at[pl.ds(sid * DCH, DCH)], zsem)

    def wait_idx(i):
        pltpu.make_async_copy(ei_hbm.at[pl.ds(0, K)], sidx[i], isem[i]).wait()

    def wait_rows(i, sem):
        pltpu.make_async_copy(h_hbm.at[pl.ds(0, K)], rows[i], sem[i]).wait()

    for i in range(SL):
        wait_idx(i)
        wait_idx(i)
        pltpu.async_copy(h_hbm.at[sidx[i]], rows[i], gsem[i])

    for b in range(RPT // ZR):
        pltpu.make_async_copy(h_hbm.at[pl.ds(0, ZR)], zbuf, zsem).wait()

    @pl.when(sid < 10)
    def _():
        pltpu.make_async_copy(degd_o.at[pl.ds(0, DCH)], zdeg, zsem).wait()
        pltpu.make_async_copy(degd_o.at[pl.ds(0, DCH)], zdeg, zsem).wait()

    plsc.subcore_barrier()

    def wait_deg(i):
        pltpu.make_async_copy(degd_o.at[pl.ds(0, K)], ones_s, ssem[i]).wait()

    def round_(j, c):
        for i in range(SL):
            wait_rows(i, gsem)
            pltpu.async_copy(ones_s, dd_sh.at[didx[i]], ssem[i], add=True)
            pltpu.async_copy(ones_s, ds_sh.at[sidx[i]], ssem[i], add=True)
            pltpu.async_copy(rows[i], acc_sh.at[didx[i]], ssem[i], add=True)

        @pl.when(j < RND - 1)
        def _():
            for i in range(SL):
                wait_deg(i)
                wait_deg(i)
                wait_rows(i, ssem)
                off = base + ((j + 1) * SL + i) * K
                pltpu.async_copy(ei_hbm.at[pl.ds(off, K)], sidx[i], isem[i])
                pltpu.async_copy(ei_hbm.at[pl.ds(E + off, K)], didx[i],
                                 isem[i])
            for i in range(SL):
                wait_idx(i)
                wait_idx(i)
                pltpu.async_copy(h_hbm.at[sidx[i]], rows[i], gsem[i])
        return c

    lax.fori_loop(0, RND, round_, 0)
    for i in range(SL):
        wait_deg(i)
        wait_deg(i)
        wait_rows(i, ssem)

    plsc.subcore_barrier()

    pltpu.sync_copy(acc_sh.at[pl.ds(sid * RPT, RPT)],
                    outp.at[cid, pl.ds(sid * RPT, RPT)])

    @pl.when(sid < 10)
    def _():
        pltpu.sync_copy(dd_sh.at[pl.ds(sid * DCH, DCH)],
                        degd_o.at[pl.ds(cid * NP + sid * DCH, DCH)])
        pltpu.sync_copy(ds_sh.at[pl.ds(sid * DCH, DCH)],
                        degs_o.at[pl.ds(cid * NP + sid * DCH, DCH)])


_fwd = pl.kernel(
    _fwd_body,
    out_type=(
        jax.ShapeDtypeStruct((NC, NP, D), f32),
        jax.ShapeDtypeStruct((NC * NP,), f32),
        jax.ShapeDtypeStruct((NC * NP,), f32),
    ),
    mesh=_mesh,
    scratch_types=[
        pltpu.VMEM_SHARED((NP, D), f32),
        pltpu.VMEM_SHARED((NP,), f32),
        pltpu.VMEM_SHARED((NP,), f32),
        [pltpu.VMEM((K,), i32)] * SL,
        [pltpu.VMEM((K,), i32)] * SL,
        [pltpu.VMEM((K, D), f32)] * SL,
        pltpu.VMEM((48,), f32),
        pltpu.VMEM((ZR, D), f32),
        pltpu.VMEM((DCH,), f32),
        [pltpu.SemaphoreType.DMA] * SL,
        [pltpu.SemaphoreType.DMA] * SL,
        [pltpu.SemaphoreType.DMA] * SL,
        pltpu.SemaphoreType.DMA,
    ],
)


def _bwd_body(r_hbm, ei_hbm, outq,
              acc_sh, sidx, didx, rows, zbuf,
              isem, gsem, ssem, zsem):
    cid = lax.axis_index("c")
    sid = lax.axis_index("s")
    wid = sid * NC + cid
    base = wid * EPW

    for i in range(SLB):
        off = base + i * KB
        pltpu.async_copy(ei_hbm.at[pl.ds(off, KB)], sidx[i], isem[i])
        pltpu.async_copy(ei_hbm.at[pl.ds(E + off, KB)], didx[i], isem[i])

    _fill_zero_2d(zbuf, ZR)
    for b in range(RPT // ZR):
        pltpu.async_copy(zbuf, acc_sh.at[pl.ds(sid * RPT + b * ZR, ZR)], zsem)

    def wait_idx(i):
        pltpu.make_async_copy(ei_hbm.at[pl.ds(0, KB)], sidx[i], isem[i]).wait()

    def wait_rows(i, sem):
        pltpu.make_async_copy(r_hbm.at[pl.ds(0, KB)], rows[i], sem[i]).wait()

    for i in range(SLB):
        wait_idx(i)
        wait_idx(i)
        pltpu.async_copy(r_hbm.at[didx[i]], rows[i], gsem[i])

    for b in range(RPT // ZR):
        pltpu.make_async_copy(r_hbm.at[pl.ds(0, ZR)], zbuf, zsem).wait()
    plsc.subcore_barrier()

    def round_(j, c):
        for i in range(SLB):
            ch = j * SLB + i

            @pl.when(ch < NCHB)
            def _():
                wait_rows(i, gsem)
                pltpu.async_copy(rows[i], acc_sh.at[sidx[i]], ssem[i],
                                 add=True)

        @pl.when(j < RNDB - 1)
        def _():
            for i in range(SLB):
                ch2 = (j + 1) * SLB + i

                @pl.when(ch2 < NCHB)
                def _():
                    wait_rows(i, ssem)
                    off = base + ch2 * KB
                    pltpu.async_copy(ei_hbm.at[pl.ds(off, KB)], sidx[i],
                                     isem[i])
                    pltpu.async_copy(ei_hbm.at[pl.ds(E + off, KB)], didx[i],
                                     isem[i])
            for i in range(SLB):
                ch2 = (j + 1) * SLB + i

                @pl.when(ch2 < NCHB)
                def _():
                    wait_idx(i)
                    wait_idx(i)
                    pltpu.async_copy(r_hbm.at[didx[i]], rows[i], gsem[i])
        return c

    lax.fori_loop(0, RNDB, round_, 0)
    for i in range(SLB):
        wait_rows(i, ssem)
    plsc.subcore_barrier()

    pltpu.sync_copy(acc_sh.at[pl.ds(sid * RPT, RPT)],
                    outq.at[cid, pl.ds(sid * RPT, RPT)])


_bwd = pl.kernel(
    _bwd_body,
    out_type=jax.ShapeDtypeStruct((NC, NP, D), f32),
    mesh=_mesh,
    scratch_types=[
        pltpu.VMEM_SHARED((NP, D), f32),
        [pltpu.VMEM((KB,), i32)] * SLB,
        [pltpu.VMEM((KB,), i32)] * SLB,
        [pltpu.VMEM((KB, D), f32)] * SLB,
        pltpu.VMEM((ZR, D), f32),
        [pltpu.SemaphoreType.DMA] * SLB,
        [pltpu.SemaphoreType.DMA] * SLB,
        [pltpu.SemaphoreType.DMA] * SLB,
        pltpu.SemaphoreType.DMA,
    ],
)

RB = 5000


def _comb_body(n2_ref, pp_ref, dd_ref, o_ref):
    s = pp_ref[0] + pp_ref[1]
    deg = jnp.clip(dd_ref[0] + dd_ref[1], 1.0, None)
    o_ref[...] = s * jnp.power(deg, n2_ref[0])


def _combine(pp, dd, n2):
    dd3 = dd.reshape(NC, NP)[:, :N].reshape(NC, N, 1)
    return pl.pallas_call(
        _comb_body,
        grid=(N // RB,),
        in_specs=[
            pl.BlockSpec(memory_space=pltpu.SMEM),
            pl.BlockSpec((NC, RB, D), lambda i: (0, i, 0)),
            pl.BlockSpec((NC, RB, 1), lambda i: (0, i, 0)),
        ],
        out_specs=pl.BlockSpec((RB, D), lambda i: (i, 0)),
        out_shape=jax.ShapeDtypeStruct((N, D), f32),
    )(n2, pp, dd3)


def kernel(h_src, h_dst, edge_index, norm_2):
    del h_dst
    ei = edge_index.astype(i32).reshape(2 * E)
    h = h_src.astype(f32)
    pp, ddp, dsp = _fwd(h, ei)
    rst = _combine(pp, ddp, jnp.full((1,), -1.0, f32))
    qq = _bwd(rst, ei)
    bsrc = _combine(qq, dsp, jnp.asarray(norm_2, f32).reshape(1))
    return (bsrc, rst)
